# baseline (speedup 1.0000x reference)
import functools

import jax
import jax.numpy as jnp
from jax import lax
from jax.experimental import pallas as pl
from jax.experimental.pallas import tpu as pltpu
from jax.experimental.pallas import tpu_sc as plsc

NUM_CORES = 2
NUM_SUBCORES = 16
LANES = 16
CHUNK = 8
NSLOT = 2


def _build_sc_kernel(B, N, T, D):
    n_workers = NUM_CORES * NUM_SUBCORES
    t_w = T // n_workers
    n_chunks = t_w // CHUNK
    n_main = (n_chunks - 2) // NSLOT * NSLOT
    vecs_per_row = D // LANES
    rows_w = B * t_w
    crows = B * CHUNK

    mesh = plsc.VectorSubcoreMesh(
        core_axis_name="c", subcore_axis_name="s",
        num_cores=NUM_CORES, num_subcores=NUM_SUBCORES)

    @functools.partial(
        pl.kernel,
        out_type=jax.ShapeDtypeStruct((N, D), jnp.float32),
        mesh=mesh,
        scratch_types=[
            pltpu.VMEM((rows_w,), jnp.int32),
            pltpu.VMEM((NSLOT, crows, D), jnp.float32),
            pltpu.VMEM((NSLOT, CHUNK, D), jnp.float32),
            [pltpu.SemaphoreType.DMA] * NSLOT,
            [pltpu.SemaphoreType.DMA] * NSLOT,
            [pltpu.SemaphoreType.DMA] * NSLOT,
        ],
    )
    def sc_kernel(xp_hbm, tok_hbm, pos_hbm, out_hbm, idx_v, gbuf, pbuf,
                  gsems, psems, osems):
        wid = lax.axis_index("s") * NUM_CORES + lax.axis_index("c")
        base_t = wid * t_w

        pltpu.sync_copy(xp_hbm.at[pl.ds(wid * rows_w, rows_w)], idx_v)

        def fire_in(j, slot):
            pltpu.async_copy(
                pos_hbm.at[pl.ds(base_t + j * CHUNK, CHUNK)], pbuf.at[slot],
                psems[slot])
            pltpu.async_copy(
                tok_hbm.at[idx_v.at[pl.ds(j * crows, crows)]],
                gbuf.at[slot], gsems[slot])

        def wait_in(j, slot):
            pltpu.make_async_copy(
                pos_hbm.at[pl.ds(base_t + j * CHUNK, CHUNK)], pbuf.at[slot],
                psems[slot]).wait()
            pltpu.make_async_copy(
                tok_hbm.at[idx_v.at[pl.ds(j * crows, crows)]],
                gbuf.at[slot], gsems[slot]).wait()

        def fire_out(j, slot):
            t_off = j * CHUNK
            for b in range(B):
                pltpu.async_copy(
                    gbuf.at[slot, pl.ds(b * CHUNK, CHUNK)],
                    out_hbm.at[pl.ds(b * T + base_t + t_off, CHUNK)],
                    osems[slot])

        def wait_out(j, slot):
            t_off = j * CHUNK
            for b in range(B):
                pltpu.make_async_copy(
                    gbuf.at[slot, pl.ds(b * CHUNK, CHUNK)],
                    out_hbm.at[pl.ds(b * T + base_t + t_off, CHUNK)],
                    osems[slot]).wait()

        def add_chunk(sl):
            def add_row(r, c):
                for col in range(vecs_per_row):
                    vsl = pl.ds(col * LANES, LANES)
                    vp = pbuf[sl, r, vsl]
                    for b in range(B):
                        gbuf[sl, b * CHUNK + r, vsl] = gbuf[sl, b * CHUNK + r, vsl] + vp
                return c
            lax.fori_loop(0, CHUNK, add_row, 0)

        def body(j, sl, prefetch):
            nxt = (sl + 1) % NSLOT
            if prefetch:
                @pl.when(j + 1 >= NSLOT)
                def _():
                    wait_out(j + 1 - NSLOT, nxt)
                fire_in(j + 1, nxt)
            wait_in(j, sl)
            add_chunk(sl)
            fire_out(j, sl)

        fire_in(0, 0)

        @pl.loop(0, n_main, step=NSLOT)
        def pipeline(jj):
            for k in range(NSLOT):
                body(jj + k, k, True)

        for j in range(n_main, n_chunks):
            body(j, j % NSLOT, j + 1 < n_chunks)

        for j in range(n_chunks - NSLOT, n_chunks):
            wait_out(j, j % NSLOT)

    return sc_kernel


def kernel(x, tok_emb, pos_emb):
    B, T = x.shape
    V, D = tok_emb.shape
    N = B * T
    n_workers = NUM_CORES * NUM_SUBCORES
    t_w = T // n_workers
    n_chunks = t_w // CHUNK
    xp = (x.reshape(B, n_workers, n_chunks, CHUNK)
          .transpose(1, 2, 0, 3).reshape(N))
    sc_kernel = _build_sc_kernel(B, N, T, D)
    out = sc_kernel(xp, tok_emb, pos_emb)
    return out.reshape(B, T, D)


# speedup vs baseline: 1.0245x; 1.0245x over previous
import functools

import jax
import jax.numpy as jnp
from jax import lax
from jax.experimental import pallas as pl
from jax.experimental.pallas import tpu as pltpu
from jax.experimental.pallas import tpu_sc as plsc

NUM_CORES = 2
NUM_SUBCORES = 16
LANES = 16
CHUNK = 8


def _build_sc_kernel(B, N, T, D):
    n_workers = NUM_CORES * NUM_SUBCORES
    t_w = T // n_workers
    n_chunks = t_w // CHUNK
    vecs_per_row = D // LANES

    mesh = plsc.VectorSubcoreMesh(
        core_axis_name="c", subcore_axis_name="s",
        num_cores=NUM_CORES, num_subcores=NUM_SUBCORES)

    @functools.partial(
        pl.kernel,
        out_type=jax.ShapeDtypeStruct((N, D), jnp.float32),
        mesh=mesh,
        scratch_types=[
            pltpu.VMEM((B * t_w,), jnp.int32),
            pltpu.VMEM((2, B, CHUNK, D), jnp.float32),
            pltpu.VMEM((2, CHUNK, D), jnp.float32),
            pltpu.SemaphoreType.DMA,
            pltpu.SemaphoreType.DMA,
            pltpu.SemaphoreType.DMA,
            pltpu.SemaphoreType.DMA,
            pltpu.SemaphoreType.DMA,
            pltpu.SemaphoreType.DMA,
        ],
    )
    def sc_kernel(x_hbm, tok_hbm, pos_hbm, out_hbm, idx_v, gbuf, pbuf,
                  gsem0, gsem1, psem0, psem1, osem0, osem1):
        wid = lax.axis_index("s") * NUM_CORES + lax.axis_index("c")
        base_t = wid * t_w
        gsems = (gsem0, gsem1)
        psems = (psem0, psem1)
        osems = (osem0, osem1)

        for b in range(B):
            pltpu.sync_copy(x_hbm.at[pl.ds(b * T + base_t, t_w)],
                            idx_v.at[pl.ds(b * t_w, t_w)])

        def fire_gathers(j, slot):
            t_off = j * CHUNK
            for b in range(B):
                pltpu.async_copy(
                    tok_hbm.at[idx_v.at[pl.ds(b * t_w + t_off, CHUNK)]],
                    gbuf.at[slot, b], gsems[slot])

        def fire_in(j, slot):
            pltpu.async_copy(
                pos_hbm.at[pl.ds(base_t + j * CHUNK, CHUNK)], pbuf.at[slot],
                psems[slot])
            fire_gathers(j, slot)

        def wait_in(j, slot):
            t_off = j * CHUNK
            pltpu.make_async_copy(
                pos_hbm.at[pl.ds(base_t + t_off, CHUNK)], pbuf.at[slot],
                psems[slot]).wait()
            for b in range(B):
                pltpu.make_async_copy(
                    tok_hbm.at[idx_v.at[pl.ds(b * t_w + t_off, CHUNK)]],
                    gbuf.at[slot, b], gsems[slot]).wait()

        def fire_out(j, slot):
            t_off = j * CHUNK
            for b in range(B):
                pltpu.async_copy(
                    gbuf.at[slot, b],
                    out_hbm.at[pl.ds(b * T + base_t + t_off, CHUNK)],
                    osems[slot])

        def wait_out(j, slot):
            t_off = j * CHUNK
            for b in range(B):
                pltpu.make_async_copy(
                    gbuf.at[slot, b],
                    out_hbm.at[pl.ds(b * T + base_t + t_off, CHUNK)],
                    osems[slot]).wait()

        fire_in(0, 0)

        @pl.loop(0, n_chunks, step=2)
        def pipeline(jj):
            for sl in (0, 1):
                j = jj + sl
                nxt = 1 - sl

                @pl.when(j + 1 < n_chunks)
                def _():
                    pltpu.async_copy(
                        pos_hbm.at[pl.ds(base_t + (j + 1) * CHUNK, CHUNK)],
                        pbuf.at[nxt], psems[nxt])

                    @pl.when(j >= 1)
                    def _():
                        wait_out(j - 1, nxt)
                    fire_gathers(j + 1, nxt)

                wait_in(j, sl)

                def add_row(r, c):
                    for col in range(vecs_per_row):
                        vsl = pl.ds(col * LANES, LANES)
                        vp = pbuf[sl, r, vsl]
                        for b in range(B):
                            gbuf[sl, b, r, vsl] = gbuf[sl, b, r, vsl] + vp
                    return c

                lax.fori_loop(0, CHUNK, add_row, 0)
                fire_out(j, sl)

        wait_out(n_chunks - 2, 0)
        wait_out(n_chunks - 1, 1)

    return sc_kernel


def kernel(x, tok_emb, pos_emb):
    B, T = x.shape
    V, D = tok_emb.shape
    N = B * T
    sc_kernel = _build_sc_kernel(B, N, T, D)
    out = sc_kernel(x.reshape(N), tok_emb, pos_emb)
    return out.reshape(B, T, D)


# speedup vs baseline: 1.0298x; 1.0051x over previous
import functools

import jax
import jax.numpy as jnp
from jax import lax
from jax.experimental import pallas as pl
from jax.experimental.pallas import tpu as pltpu
from jax.experimental.pallas import tpu_sc as plsc

NUM_CORES = 2
NUM_SUBCORES = 16
LANES = 16
CHUNK = 8


def _build_sc_kernel(B, N, T, D):
    n_workers = NUM_CORES * NUM_SUBCORES
    t_w = T // n_workers
    n_chunks = t_w // CHUNK
    vecs_per_row = D // LANES

    mesh = plsc.VectorSubcoreMesh(
        core_axis_name="c", subcore_axis_name="s",
        num_cores=NUM_CORES, num_subcores=NUM_SUBCORES)

    @functools.partial(
        pl.kernel,
        out_type=jax.ShapeDtypeStruct((N, D), jnp.float32),
        mesh=mesh,
        scratch_types=[
            pltpu.VMEM((B * t_w,), jnp.int32),
            pltpu.VMEM((2, B, CHUNK, D), jnp.float32),
            pltpu.VMEM((2, CHUNK, D), jnp.float32),
            pltpu.SemaphoreType.DMA,
            pltpu.SemaphoreType.DMA,
            pltpu.SemaphoreType.DMA,
            pltpu.SemaphoreType.DMA,
            pltpu.SemaphoreType.DMA,
            pltpu.SemaphoreType.DMA,
        ],
    )
    def sc_kernel(x_hbm, tok_hbm, pos_hbm, out_hbm, idx_v, gbuf, pbuf,
                  gsem0, gsem1, psem0, psem1, osem0, osem1):
        wid = lax.axis_index("s") * NUM_CORES + lax.axis_index("c")
        base_t = wid * t_w
        gsems = (gsem0, gsem1)
        psems = (psem0, psem1)
        osems = (osem0, osem1)

        for b in range(B):
            pltpu.sync_copy(x_hbm.at[pl.ds(b * T + base_t, t_w)],
                            idx_v.at[pl.ds(b * t_w, t_w)])

        def fire_gathers(j, slot):
            t_off = j * CHUNK
            for b in range(B):
                pltpu.async_copy(
                    tok_hbm.at[idx_v.at[pl.ds(b * t_w + t_off, CHUNK)]],
                    gbuf.at[slot, b], gsems[slot])

        def fire_in(j, slot):
            pltpu.async_copy(
                pos_hbm.at[pl.ds(base_t + j * CHUNK, CHUNK)], pbuf.at[slot],
                psems[slot])
            fire_gathers(j, slot)

        def wait_in(j, slot):
            t_off = j * CHUNK
            pltpu.make_async_copy(
                pos_hbm.at[pl.ds(base_t + t_off, CHUNK)], pbuf.at[slot],
                psems[slot]).wait()
            for b in range(B):
                pltpu.make_async_copy(
                    tok_hbm.at[idx_v.at[pl.ds(b * t_w + t_off, CHUNK)]],
                    gbuf.at[slot, b], gsems[slot]).wait()

        def fire_out(j, slot):
            t_off = j * CHUNK
            for b in range(B):
                pltpu.async_copy(
                    gbuf.at[slot, b],
                    out_hbm.at[pl.ds(b * T + base_t + t_off, CHUNK)],
                    osems[slot])

        def wait_out(j, slot):
            t_off = j * CHUNK
            for b in range(B):
                pltpu.make_async_copy(
                    gbuf.at[slot, b],
                    out_hbm.at[pl.ds(b * T + base_t + t_off, CHUNK)],
                    osems[slot]).wait()

        fire_in(0, 0)

        @pl.loop(0, n_chunks, step=2)
        def pipeline(jj):
            for sl in (0, 1):
                j = jj + sl
                nxt = 1 - sl

                @pl.when(j + 1 < n_chunks)
                def _():
                    pltpu.async_copy(
                        pos_hbm.at[pl.ds(base_t + (j + 1) * CHUNK, CHUNK)],
                        pbuf.at[nxt], psems[nxt])

                    for b in range(B):
                        @pl.when(j >= 1)
                        def _(b=b):
                            pltpu.make_async_copy(
                                gbuf.at[nxt, b],
                                out_hbm.at[pl.ds(
                                    b * T + base_t + (j - 1) * CHUNK, CHUNK)],
                                osems[nxt]).wait()
                        pltpu.async_copy(
                            tok_hbm.at[idx_v.at[
                                pl.ds(b * t_w + (j + 1) * CHUNK, CHUNK)]],
                            gbuf.at[nxt, b], gsems[nxt])

                wait_in(j, sl)

                def add_row(r, c):
                    for col in range(vecs_per_row):
                        vsl = pl.ds(col * LANES, LANES)
                        vp = pbuf[sl, r, vsl]
                        for b in range(B):
                            gbuf[sl, b, r, vsl] = gbuf[sl, b, r, vsl] + vp
                    return c

                lax.fori_loop(0, CHUNK, add_row, 0)
                fire_out(j, sl)

        wait_out(n_chunks - 2, 0)
        wait_out(n_chunks - 1, 1)

    return sc_kernel


def kernel(x, tok_emb, pos_emb):
    B, T = x.shape
    V, D = tok_emb.shape
    N = B * T
    sc_kernel = _build_sc_kernel(B, N, T, D)
    out = sc_kernel(x.reshape(N), tok_emb, pos_emb)
    return out.reshape(B, T, D)
